# baseline (speedup 1.0000x reference)
import math

import jax
import jax.numpy as jnp
from jax.experimental import pallas as pl
from jax.experimental.pallas import tpu as pltpu

_C = math.sqrt(2.0 / math.pi)
_K = _C * 0.044715

_D = 2048
_CH = 256
_SM = 64
_NBUF = 8
_A_ROWS = 2 * _NBUF * _SM
_C_ROWS = 2 * _NBUF * _SM


def _gelu(x):
    u = x * (_C + _K * (x * x))
    h = 0.5 * x
    return h + h * jnp.tanh(u)


def _pipeline_body(x_hbm, o_hbm, inb, outb, in_sem, out_sem):
    n_rows = x_hbm.shape[0]
    b_rows = n_rows - _A_ROWS - _C_ROWS
    b_chunks = b_rows // _CH
    rounds = b_chunks // _NBUF
    b0 = _A_ROWS
    c0 = _A_ROWS + b_rows

    def start_in(row, rows, slot):
        pltpu.make_async_copy(
            x_hbm.at[pl.ds(row, rows), :], inb.at[slot, pl.ds(0, rows)],
            in_sem.at[slot],
        ).start()

    def wait_in(row, rows, slot):
        pltpu.make_async_copy(
            x_hbm.at[pl.ds(row, rows), :], inb.at[slot, pl.ds(0, rows)],
            in_sem.at[slot],
        ).wait()

    def start_out(row, rows, slot):
        pltpu.make_async_copy(
            outb.at[slot, pl.ds(0, rows)], o_hbm.at[pl.ds(row, rows), :],
            out_sem.at[slot],
        ).start()

    def wait_out(row, rows, slot):
        pltpu.make_async_copy(
            outb.at[slot, pl.ds(0, rows)], o_hbm.at[pl.ds(row, rows), :],
            out_sem.at[slot],
        ).wait()

    for s in range(_NBUF):
        start_in(s * _SM, _SM, s)
    for c in range(_NBUF):
        wait_in(c * _SM, _SM, c)
        outb[c, :_SM] = _gelu(inb[c, :_SM])
        start_out(c * _SM, _SM, c)
        start_in((c + _NBUF) * _SM, _SM, c)
    for c in range(_NBUF, 2 * _NBUF):
        s = c - _NBUF
        wait_in(c * _SM, _SM, s)
        wait_out(s * _SM, _SM, s)
        outb[s, :_SM] = _gelu(inb[s, :_SM])
        start_out(c * _SM, _SM, s)
        start_in(b0 + s * _CH, _CH, s)

    def round_body(r, _):
        for s in range(_NBUF):
            chunk = r * _NBUF + s
            row = b0 + chunk * _CH
            wait_in(row, _CH, s)

            @pl.when(r == 0)
            def _():
                wait_out((s + _NBUF) * _SM, _SM, s)

            @pl.when(r > 0)
            def _():
                wait_out(row - _NBUF * _CH, _CH, s)

            outb[s] = _gelu(inb[s])
            start_out(row, _CH, s)

            @pl.when(r < rounds - 1)
            def _():
                start_in(row + _NBUF * _CH, _CH, s)

        return 0

    jax.lax.fori_loop(0, rounds, round_body, 0)

    for s in range(_NBUF):
        start_in(c0 + s * _SM, _SM, s)
    for c in range(_NBUF):
        wait_in(c0 + c * _SM, _SM, c)
        wait_out(b0 + ((rounds - 1) * _NBUF + c) * _CH, _CH, c)
        outb[c, :_SM] = _gelu(inb[c, :_SM])
        start_out(c0 + c * _SM, _SM, c)
        start_in(c0 + (c + _NBUF) * _SM, _SM, c)
    for c in range(_NBUF, 2 * _NBUF):
        s = c - _NBUF
        wait_in(c0 + c * _SM, _SM, s)
        wait_out(c0 + s * _SM, _SM, s)
        outb[s, :_SM] = _gelu(inb[s, :_SM])
        start_out(c0 + c * _SM, _SM, s)
    for c in range(_NBUF, 2 * _NBUF):
        s = c - _NBUF
        wait_out(c0 + c * _SM, _SM, s)


def kernel(x, log_k_blend):
    B, T, D = x.shape
    R = B * T
    x2 = x.reshape(R, D)
    out = pl.pallas_call(
        _pipeline_body,
        in_specs=[pl.BlockSpec(memory_space=pltpu.HBM)],
        out_specs=pl.BlockSpec(memory_space=pltpu.HBM),
        out_shape=jax.ShapeDtypeStruct((R, D), x.dtype),
        scratch_shapes=[
            pltpu.VMEM((_NBUF, _CH, _D), jnp.float32),
            pltpu.VMEM((_NBUF, _CH, _D), jnp.float32),
            pltpu.SemaphoreType.DMA((_NBUF,)),
            pltpu.SemaphoreType.DMA((_NBUF,)),
        ],
    )(x2)
    return out.reshape(B, T, D)


# speedup vs baseline: 1.0118x; 1.0118x over previous
import math

import jax
import jax.numpy as jnp
from jax.experimental import pallas as pl
from jax.experimental.pallas import tpu as pltpu

_C = math.sqrt(2.0 / math.pi)
_K = _C * 0.044715

_D = 2048
_CH = 256
_SM = 128
_NBUF = 8
_A_ROWS = 2 * _NBUF * _SM
_C_ROWS = 2 * _NBUF * _SM


def _gelu(x):
    u = x * (_C + _K * (x * x))
    h = 0.5 * x
    return h + h * jnp.tanh(u)


def _pipeline_body(x_hbm, o_hbm, inb, outb, in_sem, out_sem):
    n_rows = x_hbm.shape[0]
    b_rows = n_rows - _A_ROWS - _C_ROWS
    b_chunks = b_rows // _CH
    rounds = b_chunks // _NBUF
    b0 = _A_ROWS
    c0 = _A_ROWS + b_rows

    def start_in(row, rows, slot):
        pltpu.make_async_copy(
            x_hbm.at[pl.ds(row, rows), :], inb.at[slot, pl.ds(0, rows)],
            in_sem.at[slot],
        ).start()

    def wait_in(row, rows, slot):
        pltpu.make_async_copy(
            x_hbm.at[pl.ds(row, rows), :], inb.at[slot, pl.ds(0, rows)],
            in_sem.at[slot],
        ).wait()

    def start_out(row, rows, slot):
        pltpu.make_async_copy(
            outb.at[slot, pl.ds(0, rows)], o_hbm.at[pl.ds(row, rows), :],
            out_sem.at[slot],
        ).start()

    def wait_out(row, rows, slot):
        pltpu.make_async_copy(
            outb.at[slot, pl.ds(0, rows)], o_hbm.at[pl.ds(row, rows), :],
            out_sem.at[slot],
        ).wait()

    for s in range(_NBUF):
        start_in(s * _SM, _SM, s)
    for c in range(_NBUF):
        wait_in(c * _SM, _SM, c)
        outb[c, :_SM] = _gelu(inb[c, :_SM])
        start_out(c * _SM, _SM, c)
        start_in((c + _NBUF) * _SM, _SM, c)
    for c in range(_NBUF, 2 * _NBUF):
        s = c - _NBUF
        wait_in(c * _SM, _SM, s)
        wait_out(s * _SM, _SM, s)
        outb[s, :_SM] = _gelu(inb[s, :_SM])
        start_out(c * _SM, _SM, s)
        start_in(b0 + s * _CH, _CH, s)

    def round_body(r, _):
        for s in range(_NBUF):
            chunk = r * _NBUF + s
            row = b0 + chunk * _CH
            wait_in(row, _CH, s)

            @pl.when(r == 0)
            def _():
                wait_out((s + _NBUF) * _SM, _SM, s)

            @pl.when(r > 0)
            def _():
                wait_out(row - _NBUF * _CH, _CH, s)

            outb[s] = _gelu(inb[s])
            start_out(row, _CH, s)

            @pl.when(r < rounds - 1)
            def _():
                start_in(row + _NBUF * _CH, _CH, s)

        return 0

    jax.lax.fori_loop(0, rounds, round_body, 0)

    for s in range(_NBUF):
        start_in(c0 + s * _SM, _SM, s)
    for c in range(_NBUF):
        wait_in(c0 + c * _SM, _SM, c)
        wait_out(b0 + ((rounds - 1) * _NBUF + c) * _CH, _CH, c)
        outb[c, :_SM] = _gelu(inb[c, :_SM])
        start_out(c0 + c * _SM, _SM, c)
        start_in(c0 + (c + _NBUF) * _SM, _SM, c)
    for c in range(_NBUF, 2 * _NBUF):
        s = c - _NBUF
        wait_in(c0 + c * _SM, _SM, s)
        wait_out(c0 + s * _SM, _SM, s)
        outb[s, :_SM] = _gelu(inb[s, :_SM])
        start_out(c0 + c * _SM, _SM, s)
    for c in range(_NBUF, 2 * _NBUF):
        s = c - _NBUF
        wait_out(c0 + c * _SM, _SM, s)


def kernel(x, log_k_blend):
    B, T, D = x.shape
    R = B * T
    x2 = x.reshape(R, D)
    out = pl.pallas_call(
        _pipeline_body,
        in_specs=[pl.BlockSpec(memory_space=pltpu.HBM)],
        out_specs=pl.BlockSpec(memory_space=pltpu.HBM),
        out_shape=jax.ShapeDtypeStruct((R, D), x.dtype),
        scratch_shapes=[
            pltpu.VMEM((_NBUF, _CH, _D), jnp.float32),
            pltpu.VMEM((_NBUF, _CH, _D), jnp.float32),
            pltpu.SemaphoreType.DMA((_NBUF,)),
            pltpu.SemaphoreType.DMA((_NBUF,)),
        ],
    )(x2)
    return out.reshape(B, T, D)


# speedup vs baseline: 1.0443x; 1.0322x over previous
import math

import jax
import jax.numpy as jnp
from jax.experimental import pallas as pl
from jax.experimental.pallas import tpu as pltpu

_C = math.sqrt(2.0 / math.pi)
_K = _C * 0.044715

_D = 2048
_CH = 512
_NBUF = 4


def _gelu(x):
    u = x * (_C + _K * (x * x))
    h = 0.5 * x
    return h + h * jnp.tanh(u)


def _pipeline_body(x_hbm, o_hbm, inb, outb, in_sem, out_sem):
    n_rows = x_hbm.shape[0]
    nchunks = n_rows // _CH
    rounds = nchunks // _NBUF

    def start_in(chunk, slot):
        pltpu.make_async_copy(
            x_hbm.at[pl.ds(chunk * _CH, _CH), :], inb.at[slot], in_sem.at[slot]
        ).start()

    def wait_in(chunk, slot):
        pltpu.make_async_copy(
            x_hbm.at[pl.ds(chunk * _CH, _CH), :], inb.at[slot], in_sem.at[slot]
        ).wait()

    def start_out(chunk, slot):
        pltpu.make_async_copy(
            outb.at[slot], o_hbm.at[pl.ds(chunk * _CH, _CH), :], out_sem.at[slot]
        ).start()

    def wait_out(chunk, slot):
        pltpu.make_async_copy(
            outb.at[slot], o_hbm.at[pl.ds(chunk * _CH, _CH), :], out_sem.at[slot]
        ).wait()

    for s in range(_NBUF):
        start_in(s, s)

    def round_body(r, _):
        for s in range(_NBUF):
            chunk = r * _NBUF + s
            wait_in(chunk, s)

            @pl.when(r > 0)
            def _():
                wait_out(chunk - _NBUF, s)

            outb[s] = _gelu(inb[s])
            start_out(chunk, s)

            @pl.when(r < rounds - 1)
            def _():
                start_in(chunk + _NBUF, s)

        return 0

    jax.lax.fori_loop(0, rounds, round_body, 0)

    for s in range(_NBUF):
        wait_out(nchunks - _NBUF + s, s)


def kernel(x, log_k_blend):
    B, T, D = x.shape
    R = B * T
    x2 = x.reshape(R, D)
    out = pl.pallas_call(
        _pipeline_body,
        in_specs=[pl.BlockSpec(memory_space=pltpu.HBM)],
        out_specs=pl.BlockSpec(memory_space=pltpu.HBM),
        out_shape=jax.ShapeDtypeStruct((R, D), x.dtype),
        scratch_shapes=[
            pltpu.VMEM((_NBUF, _CH, _D), jnp.float32),
            pltpu.VMEM((_NBUF, _CH, _D), jnp.float32),
            pltpu.SemaphoreType.DMA((_NBUF,)),
            pltpu.SemaphoreType.DMA((_NBUF,)),
        ],
    )(x2)
    return out.reshape(B, T, D)
